# baseline (speedup 1.0000x reference)
import functools

import jax
import jax.numpy as jnp
from jax import lax
from jax.experimental import pallas as pl
from jax.experimental.pallas import tpu as pltpu
from jax.experimental.pallas import tpu_sc as plsc

N = 10000
E = 320000
D = 128
NC = 2
NS = 16
NW = NC * NS
CH = 128
NCH = 80
EPW = NCH * CH
E2 = NW * EPW
GRP = 8
NGRP = NCH // GRP
NP = 10240
RPT = NP // NS


@functools.cache
def _mesh():
    return plsc.VectorSubcoreMesh(core_axis_name="c", subcore_axis_name="s")


def _deg_sc(col3d, zeros1, ones1):
    WIN = 8

    @functools.partial(
        pl.kernel, mesh=_mesh(),
        out_type=jax.ShapeDtypeStruct((NC * NP,), jnp.float32),
        scratch_types=[
            pltpu.VMEM((NCH, CH), jnp.int32),
            pltpu.VMEM((CH,), jnp.float32),
            pltpu.VMEM_SHARED((NP,), jnp.float32),
            pltpu.SemaphoreType.DMA,
        ])
    def k(col_hbm, z_hbm, ones_hbm, out_hbm, cidx, ones_v, acc, sem):
        c = lax.axis_index("c")
        s = lax.axis_index("s")
        w = c * NS + s
        zoff = pl.multiple_of(s * RPT, 8)
        pltpu.sync_copy(z_hbm.at[pl.ds(zoff, RPT)], acc.at[pl.ds(zoff, RPT)])
        pltpu.sync_copy(ones_hbm, ones_v)
        pltpu.sync_copy(col_hbm.at[w], cidx)
        plsc.subcore_barrier()

        for i in range(WIN):
            pltpu.async_copy(ones_v, acc.at[cidx.at[i]], sem, add=True)

        @pl.loop(0, NCH - WIN)
        def _(i):
            pltpu.make_async_copy(ones_v, acc.at[cidx.at[i]], sem).wait()
            pltpu.async_copy(ones_v, acc.at[cidx.at[i + WIN]], sem, add=True)

        for i in range(NCH - WIN, NCH):
            pltpu.make_async_copy(ones_v, acc.at[cidx.at[i]], sem).wait()

        plsc.subcore_barrier()
        doff = pl.multiple_of(c * NP + s * RPT, 8)
        pltpu.sync_copy(acc.at[pl.ds(zoff, RPT)], out_hbm.at[pl.ds(doff, RPT)])

    return k(col3d, zeros1, ones1)


def _scatter_sc(h, row3d, col4d, zerosD):
    @functools.partial(
        pl.kernel, mesh=_mesh(),
        out_type=jax.ShapeDtypeStruct((NC * NP, D), jnp.float32),
        scratch_types=[
            pltpu.VMEM((NCH, CH), jnp.int32),
            pltpu.VMEM((GRP, CH), jnp.int32),
            pltpu.VMEM((GRP, CH), jnp.int32),
            pltpu.VMEM((CH, D), jnp.float32),
            pltpu.VMEM((CH, D), jnp.float32),
            pltpu.VMEM_SHARED((NP, D), jnp.float32),
            pltpu.SemaphoreType.DMA,
            pltpu.SemaphoreType.DMA,
            pltpu.SemaphoreType.DMA,
            pltpu.SemaphoreType.DMA,
        ])
    def k(h_hbm, row_hbm, col_hbm, z_hbm, out_hbm, ridx, ca, cb,
          rows_a, rows_b, acc, sem_a, sem_b, sem_ca, sem_cb):
        c = lax.axis_index("c")
        s = lax.axis_index("s")
        w = c * NS + s
        zoff = pl.multiple_of(s * RPT, 8)
        pltpu.sync_copy(z_hbm.at[pl.ds(zoff, RPT)], acc.at[pl.ds(zoff, RPT)])
        pltpu.sync_copy(row_hbm.at[w], ridx)
        plsc.subcore_barrier()

        bufs = (rows_a, rows_b)
        sems = (sem_a, sem_b)
        cbufs = (ca, cb)
        csems = (sem_ca, sem_cb)

        def gat(i, p):
            pltpu.async_copy(h_hbm.at[ridx.at[i]], bufs[p], sems[p])

        def gwait(i, p):
            pltpu.make_async_copy(h_hbm.at[ridx.at[i]], bufs[p],
                                  sems[p]).wait()

        def cload(g, q):
            pltpu.async_copy(col_hbm.at[w, g], cbufs[q], csems[q])

        def cwait(g, q):
            pltpu.make_async_copy(col_hbm.at[w, g], cbufs[q], csems[q]).wait()

        cload(0, 0)
        cload(1, 1)
        gat(0, 0)

        @pl.loop(0, NGRP, step=2)
        def _(g):
            i0 = g * GRP
            cwait(g, 0)
            for j in range(GRP):
                i = i0 + j
                p = j % 2
                gat(i + 1, 1 - p)
                gwait(i, p)
                pltpu.sync_copy(bufs[p], acc.at[ca.at[j]], add=True)

            @pl.when(g + 2 < NGRP)
            def _():
                cload(g + 2, 0)

            cwait(g + 1, 1)
            for j in range(GRP):
                i = i0 + GRP + j
                p = j % 2
                gat((i + 1) % NCH, 1 - p)
                gwait(i, p)
                pltpu.sync_copy(bufs[p], acc.at[cb.at[j]], add=True)

            @pl.when(g + 3 < NGRP)
            def _():
                cload(g + 3, 1)

        gwait(0, 0)

        plsc.subcore_barrier()
        doff = pl.multiple_of(c * NP + s * RPT, 8)
        pltpu.sync_copy(acc.at[pl.ds(zoff, RPT)], out_hbm.at[pl.ds(doff, RPT)])

    return k(h, row3d, col4d, zerosD)


def _g_tc(deg2):
    def body(d_ref, o_ref):
        dsum = d_ref[0, :] + d_ref[1, :]
        g = jnp.where(dsum > 0, lax.rsqrt(dsum), 0.0)
        o_ref[...] = jnp.broadcast_to(g[:, None], (NP, D))

    return pl.pallas_call(
        body,
        grid=(1,),
        in_specs=[pl.BlockSpec((NC, NP), lambda i: (0, 0))],
        out_specs=pl.BlockSpec((NP, D), lambda i: (0, 0)),
        out_shape=jax.ShapeDtypeStruct((NP, D), jnp.float32),
    )(deg2)


def _mm_scale_tc(x, W, gb):
    BLK = 1000

    def body(x_ref, w_ref, g_ref, o_ref):
        h = lax.dot_general(x_ref[...], w_ref[...], (((1,), (1,)), ((), ())),
                            preferred_element_type=jnp.float32)
        o_ref[...] = h * g_ref[...]

    return pl.pallas_call(
        body,
        grid=(N // BLK,),
        in_specs=[
            pl.BlockSpec((BLK, D), lambda i: (i, 0)),
            pl.BlockSpec((D, D), lambda i: (0, 0)),
            pl.BlockSpec((BLK, D), lambda i: (i, 0)),
        ],
        out_specs=pl.BlockSpec((BLK, D), lambda i: (i, 0)),
        out_shape=jax.ShapeDtypeStruct((N, D), jnp.float32),
    )(x, W, gb)


def _bn_mm_tc(spair, gb, b0r, W1):
    def body(sp_ref, g_ref, b_ref, w_ref, o_ref):
        h = (sp_ref[0] + sp_ref[1]) * g_ref[...] + b_ref[...]
        m = jnp.mean(h, axis=0)
        hc = h - m
        v = jnp.mean(hc * hc, axis=0)
        hbn = jnp.maximum(hc * lax.rsqrt(v + 1e-5), 0.0)
        o_ref[...] = lax.dot_general(
            hbn, w_ref[...], (((1,), (1,)), ((), ())),
            preferred_element_type=jnp.float32) * g_ref[...]

    return pl.pallas_call(
        body,
        grid=(1,),
        in_specs=[
            pl.BlockSpec((NC, N, D), lambda i: (0, 0, 0)),
            pl.BlockSpec((N, D), lambda i: (0, 0)),
            pl.BlockSpec((1, D), lambda i: (0, 0)),
            pl.BlockSpec((D, D), lambda i: (0, 0)),
        ],
        out_specs=pl.BlockSpec((N, D), lambda i: (0, 0)),
        out_shape=jax.ShapeDtypeStruct((N, D), jnp.float32),
    )(spair, gb, b0r, W1)


def _final_tc(tpair, gb, b1r):
    BLK = 1000

    def body(tp_ref, g_ref, b_ref, o_ref):
        o_ref[...] = (tp_ref[0] + tp_ref[1]) * g_ref[...] + b_ref[...]

    return pl.pallas_call(
        body,
        grid=(N // BLK,),
        in_specs=[
            pl.BlockSpec((2, BLK, D), lambda i: (0, i, 0)),
            pl.BlockSpec((BLK, D), lambda i: (i, 0)),
            pl.BlockSpec((1, D), lambda i: (0, 0)),
        ],
        out_specs=pl.BlockSpec((BLK, D), lambda i: (i, 0)),
        out_shape=jax.ShapeDtypeStruct((N, D), jnp.float32),
    )(tpair, gb, b1r)


def kernel(x, edge_index, W0, b0, W1, b1):
    ei = edge_index.astype(jnp.int32)
    npad = E2 - E
    row_pad = jnp.concatenate([ei[0], jnp.zeros((npad,), jnp.int32)])
    col_pad = jnp.concatenate(
        [ei[1], N + (jnp.arange(npad, dtype=jnp.int32) % (NP - N))])
    row3d = row_pad.reshape(NW, NCH, CH)
    col3d = col_pad.reshape(NW, NCH, CH)
    col4d = col_pad.reshape(NW, NGRP, GRP, CH)
    zerosD = jnp.zeros((NP, D), jnp.float32)
    zeros1 = jnp.zeros((NP,), jnp.float32)
    ones1 = jnp.ones((CH,), jnp.float32)
    b0r = b0.reshape(1, D)
    b1r = b1.reshape(1, D)

    deg2 = _deg_sc(col3d, zeros1, ones1).reshape(NC, NP)
    gb = _g_tc(deg2)
    hs = _mm_scale_tc(x, W0, gb)
    spair = _scatter_sc(hs, row3d, col4d, zerosD).reshape(NC, NP, D)
    hs2 = _bn_mm_tc(spair, gb, b0r, W1)
    tpair = _scatter_sc(hs2, row3d, col4d, zerosD).reshape(NC, NP, D)
    return _final_tc(tpair, gb, b1r)


# speedup vs baseline: 3.4943x; 3.4943x over previous
import functools

import jax
import jax.numpy as jnp
from jax import lax
from jax.experimental import pallas as pl
from jax.experimental.pallas import tpu as pltpu
from jax.experimental.pallas import tpu_sc as plsc

N = 10000
E = 320000
D = 128
NC = 2
NS = 16
NW = NC * NS
CH = 128
NCH = 80
EPW = NCH * CH
E2 = NW * EPW
GRP = 8
NGRP = NCH // GRP
NP = 10240
RPT = NP // NS


@functools.cache
def _mesh():
    return plsc.VectorSubcoreMesh(core_axis_name="c", subcore_axis_name="s")


def _deg_sc(col3d, zeros1, ones1):
    WIN = 8

    @functools.partial(
        pl.kernel, mesh=_mesh(),
        out_type=jax.ShapeDtypeStruct((NC * NP,), jnp.float32),
        scratch_types=[
            pltpu.VMEM((NCH, CH), jnp.int32),
            pltpu.VMEM((CH,), jnp.float32),
            pltpu.VMEM_SHARED((NP,), jnp.float32),
            pltpu.SemaphoreType.DMA,
        ])
    def k(col_hbm, z_hbm, ones_hbm, out_hbm, cidx, ones_v, acc, sem):
        c = lax.axis_index("c")
        s = lax.axis_index("s")
        w = c * NS + s
        zoff = pl.multiple_of(s * RPT, 8)
        pltpu.sync_copy(z_hbm.at[pl.ds(zoff, RPT)], acc.at[pl.ds(zoff, RPT)])
        pltpu.sync_copy(ones_hbm, ones_v)
        pltpu.sync_copy(col_hbm.at[w], cidx)
        plsc.subcore_barrier()

        for i in range(WIN):
            pltpu.async_copy(ones_v, acc.at[cidx.at[i]], sem, add=True)

        @pl.loop(0, NCH - WIN)
        def _(i):
            pltpu.make_async_copy(ones_v, acc.at[cidx.at[i]], sem).wait()
            pltpu.async_copy(ones_v, acc.at[cidx.at[i + WIN]], sem, add=True)

        for i in range(NCH - WIN, NCH):
            pltpu.make_async_copy(ones_v, acc.at[cidx.at[i]], sem).wait()

        plsc.subcore_barrier()
        doff = pl.multiple_of(c * NP + s * RPT, 8)
        pltpu.sync_copy(acc.at[pl.ds(zoff, RPT)], out_hbm.at[pl.ds(doff, RPT)])

    return k(col3d, zeros1, ones1)


def _scatter_sc(h, row3d, col4d, zerosD):
    @functools.partial(
        pl.kernel, mesh=_mesh(),
        out_type=jax.ShapeDtypeStruct((NC * NP, D), jnp.float32),
        scratch_types=[
            pltpu.VMEM((NCH, CH), jnp.int32),
            pltpu.VMEM((GRP, CH), jnp.int32),
            pltpu.VMEM((GRP, CH), jnp.int32),
            pltpu.VMEM((CH, D), jnp.float32),
            pltpu.VMEM((CH, D), jnp.float32),
            pltpu.VMEM_SHARED((NP, D), jnp.float32),
            pltpu.SemaphoreType.DMA,
            pltpu.SemaphoreType.DMA,
            pltpu.SemaphoreType.DMA,
            pltpu.SemaphoreType.DMA,
        ])
    def k(h_hbm, row_hbm, col_hbm, z_hbm, out_hbm, ridx, ca, cb,
          rows_a, rows_b, acc, sem_a, sem_b, sem_ca, sem_cb):
        c = lax.axis_index("c")
        s = lax.axis_index("s")
        w = c * NS + s
        zoff = pl.multiple_of(s * RPT, 8)
        pltpu.sync_copy(z_hbm.at[pl.ds(zoff, RPT)], acc.at[pl.ds(zoff, RPT)])
        pltpu.sync_copy(row_hbm.at[w], ridx)
        plsc.subcore_barrier()

        bufs = (rows_a, rows_b)
        sems = (sem_a, sem_b)
        cbufs = (ca, cb)
        csems = (sem_ca, sem_cb)

        def gat(i, p):
            pltpu.async_copy(h_hbm.at[ridx.at[i]], bufs[p], sems[p])

        def gwait(i, p):
            pltpu.make_async_copy(h_hbm.at[ridx.at[i]], bufs[p],
                                  sems[p]).wait()

        def cload(g, q):
            pltpu.async_copy(col_hbm.at[w, g], cbufs[q], csems[q])

        def cwait(g, q):
            pltpu.make_async_copy(col_hbm.at[w, g], cbufs[q], csems[q]).wait()

        cload(0, 0)
        cload(1, 1)
        gat(0, 0)

        @pl.loop(0, NGRP, step=2)
        def _(g):
            i0 = g * GRP
            cwait(g, 0)
            for j in range(GRP):
                i = i0 + j
                p = j % 2
                gat(i + 1, 1 - p)
                gwait(i, p)
                pltpu.sync_copy(bufs[p], acc.at[ca.at[j]], add=True)

            @pl.when(g + 2 < NGRP)
            def _():
                cload(g + 2, 0)

            cwait(g + 1, 1)
            for j in range(GRP):
                i = i0 + GRP + j
                p = j % 2
                gat((i + 1) % NCH, 1 - p)
                gwait(i, p)
                pltpu.sync_copy(bufs[p], acc.at[cb.at[j]], add=True)

            @pl.when(g + 3 < NGRP)
            def _():
                cload(g + 3, 1)

        gwait(0, 0)

        plsc.subcore_barrier()
        doff = pl.multiple_of(c * NP + s * RPT, 8)
        pltpu.sync_copy(acc.at[pl.ds(zoff, RPT)], out_hbm.at[pl.ds(doff, RPT)])

    return k(h, row3d, col4d, zerosD)


def _g_tc(deg2):
    def body(d_ref, o_ref):
        dsum = d_ref[0, :] + d_ref[1, :]
        g = jnp.where(dsum > 0, lax.rsqrt(dsum), 0.0)
        o_ref[...] = jnp.broadcast_to(g[:, None], (NP, D))

    return pl.pallas_call(
        body,
        grid=(1,),
        in_specs=[pl.BlockSpec((NC, NP), lambda i: (0, 0))],
        out_specs=pl.BlockSpec((NP, D), lambda i: (0, 0)),
        out_shape=jax.ShapeDtypeStruct((NP, D), jnp.float32),
    )(deg2)


def _mm_scale_tc(x, W, gb):
    BLK = 1000

    def body(x_ref, w_ref, g_ref, o_ref):
        h = lax.dot_general(x_ref[...], w_ref[...], (((1,), (1,)), ((), ())),
                            preferred_element_type=jnp.float32)
        o_ref[...] = h * g_ref[...]

    return pl.pallas_call(
        body,
        grid=(N // BLK,),
        in_specs=[
            pl.BlockSpec((BLK, D), lambda i: (i, 0)),
            pl.BlockSpec((D, D), lambda i: (0, 0)),
            pl.BlockSpec((BLK, D), lambda i: (i, 0)),
        ],
        out_specs=pl.BlockSpec((BLK, D), lambda i: (i, 0)),
        out_shape=jax.ShapeDtypeStruct((N, D), jnp.float32),
    )(x, W, gb)


def _bn_mm_tc(spair, gb, b0r, W1):
    def body(sp_ref, g_ref, b_ref, w_ref, o_ref):
        h = (sp_ref[0] + sp_ref[1]) * g_ref[...] + b_ref[...]
        m = jnp.mean(h, axis=0)
        hc = h - m
        v = jnp.mean(hc * hc, axis=0)
        hbn = jnp.maximum(hc * lax.rsqrt(v + 1e-5), 0.0)
        o_ref[...] = lax.dot_general(
            hbn, w_ref[...], (((1,), (1,)), ((), ())),
            preferred_element_type=jnp.float32) * g_ref[...]

    return pl.pallas_call(
        body,
        grid=(1,),
        in_specs=[
            pl.BlockSpec((NC, N, D), lambda i: (0, 0, 0)),
            pl.BlockSpec((N, D), lambda i: (0, 0)),
            pl.BlockSpec((1, D), lambda i: (0, 0)),
            pl.BlockSpec((D, D), lambda i: (0, 0)),
        ],
        out_specs=pl.BlockSpec((N, D), lambda i: (0, 0)),
        out_shape=jax.ShapeDtypeStruct((N, D), jnp.float32),
    )(spair, gb, b0r, W1)


def _final_tc(tpair, gb, b1r):
    BLK = 1000

    def body(tp_ref, g_ref, b_ref, o_ref):
        o_ref[...] = (tp_ref[0] + tp_ref[1]) * g_ref[...] + b_ref[...]

    return pl.pallas_call(
        body,
        grid=(N // BLK,),
        in_specs=[
            pl.BlockSpec((2, BLK, D), lambda i: (0, i, 0)),
            pl.BlockSpec((BLK, D), lambda i: (i, 0)),
            pl.BlockSpec((1, D), lambda i: (0, 0)),
        ],
        out_specs=pl.BlockSpec((BLK, D), lambda i: (i, 0)),
        out_shape=jax.ShapeDtypeStruct((N, D), jnp.float32),
    )(tpair, gb, b1r)


def kernel(x, edge_index, W0, b0, W1, b1):
    ei = edge_index.astype(jnp.int32)
    npad = E2 - E
    row_pad = jnp.concatenate(
        [ei[0], (jnp.arange(npad, dtype=jnp.int32) * 131) % N])
    col_pad = jnp.concatenate(
        [ei[1], N + (jnp.arange(npad, dtype=jnp.int32) % (NP - N))])
    row3d = row_pad.reshape(NW, NCH, CH)
    col3d = col_pad.reshape(NW, NCH, CH)
    col4d = col_pad.reshape(NW, NGRP, GRP, CH)
    zerosD = jnp.zeros((NP, D), jnp.float32)
    zeros1 = jnp.zeros((NP,), jnp.float32)
    ones1 = jnp.ones((CH,), jnp.float32)
    b0r = b0.reshape(1, D)
    b1r = b1.reshape(1, D)

    deg2 = _deg_sc(col3d, zeros1, ones1).reshape(NC, NP)
    gb = _g_tc(deg2)
    hs = _mm_scale_tc(x, W0, gb)
    spair = _scatter_sc(hs, row3d, col4d, zerosD).reshape(NC, NP, D)
    hs2 = _bn_mm_tc(spair, gb, b0r, W1)
    tpair = _scatter_sc(hs2, row3d, col4d, zerosD).reshape(NC, NP, D)
    return _final_tc(tpair, gb, b1r)
